# baseline (speedup 1.0000x reference)
import functools

import jax
import jax.numpy as jnp
from jax import lax
from jax.experimental import pallas as pl
from jax.experimental.pallas import tpu as pltpu
from jax.experimental.pallas import tpu_sc as plsc

B = 16384
D = 32
NC = 2
NS = 16
NW = NC * NS
BPW = B // NW
CHUNK = 128
NCHUNK = BPW // CHUNK

TC_BLK = 2048
TC_GRID = B // TC_BLK


def _gather_body(user_hbm, item_hbm, theta_hbm, a_hbm, b_hbm,
                 th_out, av_out, bv_out,
                 uidx, iidx, th, av, bv, sem):
    wid = lax.axis_index("s") * NC + lax.axis_index("c")
    base = wid * BPW

    pltpu.sync_copy(user_hbm.at[wid], uidx)
    pltpu.sync_copy(item_hbm.at[wid], iidx)

    copies = []
    for j in range(NCHUNK):
        sl = pl.ds(j * CHUNK, CHUNK)
        copies.append(pltpu.async_copy(theta_hbm.at[uidx.at[j]], th.at[sl], sem))
        copies.append(pltpu.async_copy(a_hbm.at[iidx.at[j]], av.at[sl], sem))
        copies.append(pltpu.async_copy(b_hbm.at[iidx.at[j]], bv.at[sl], sem))
    for c in copies:
        c.wait()

    pltpu.sync_copy(th, th_out.at[pl.ds(base, BPW)])
    pltpu.sync_copy(av, av_out.at[pl.ds(base, BPW)])
    pltpu.sync_copy(bv, bv_out.at[pl.ds(base, BPW)])


def _compute_body(th_ref, av_ref, bv_ref, out_ref):
    t = th_ref[...]
    a = av_ref[...]
    b = bv_ref[...]
    sa = jax.nn.sigmoid(a)
    s = jnp.sum(sa * t, axis=-1)
    out_ref[...] = jax.nn.sigmoid(s.reshape(b.shape) - b)


@jax.jit
def _mirt(user, item, theta_w, a_w, b_flat):
    mesh = plsc.VectorSubcoreMesh(core_axis_name="c", subcore_axis_name="s",
                                  num_cores=NC, num_subcores=NS)
    gather = functools.partial(
        pl.kernel,
        out_type=(
            jax.ShapeDtypeStruct((B, D), jnp.float32),
            jax.ShapeDtypeStruct((B, D), jnp.float32),
            jax.ShapeDtypeStruct((B,), jnp.float32),
        ),
        mesh=mesh,
        compiler_params=pltpu.CompilerParams(use_tc_tiling_on_sc=False),
        scratch_types=[
            pltpu.VMEM((NCHUNK, CHUNK), jnp.int32),
            pltpu.VMEM((NCHUNK, CHUNK), jnp.int32),
            pltpu.VMEM((BPW, D), jnp.float32),
            pltpu.VMEM((BPW, D), jnp.float32),
            pltpu.VMEM((BPW,), jnp.float32),
            pltpu.SemaphoreType.DMA,
        ],
    )(_gather_body)
    th, av, bv = gather(user.reshape(NW, NCHUNK, CHUNK),
                        item.reshape(NW, NCHUNK, CHUNK),
                        theta_w, a_w, b_flat)

    out = pl.pallas_call(
        _compute_body,
        grid=(TC_GRID,),
        in_specs=[
            pl.BlockSpec((TC_BLK, D), lambda i: (i, 0)),
            pl.BlockSpec((TC_BLK, D), lambda i: (i, 0)),
            pl.BlockSpec((8, TC_BLK // 8), lambda i: (i, 0)),
        ],
        out_specs=pl.BlockSpec((8, TC_BLK // 8), lambda i: (i, 0)),
        out_shape=jax.ShapeDtypeStruct((TC_GRID * 8, TC_BLK // 8), jnp.float32),
    )(th, av, bv.reshape(TC_GRID * 8, TC_BLK // 8))
    return out.reshape(B)


def kernel(user, item, theta_w, a_w, b_w):
    return _mirt(user, item, theta_w, a_w, b_w.reshape(-1))
